# baseline (device time: 63560 ns/iter reference)
import functools

import jax
import jax.numpy as jnp
from jax import lax
from jax.experimental import pallas as pl
from jax.experimental.pallas import tpu as pltpu

N_DEV = 32
N_ROUNDS = 5
B, Sq, Hq, Dh = 2, 128, 4, 64
SKV_LOC = 128
NEG = -1e9


def kernel(x, Wq, K_ext, V_ext, Wo):
    def body(x_ref, wq_ref, k_ref, v_ref, wo_ref, out_ref,
             acc_ref, ml_ref, comm_acc, comm_ml, send_sems, recv_sems):
        my_pos = lax.axis_index("i")

        barrier_sem = pltpu.get_barrier_semaphore()
        for k in range(N_ROUNDS):
            partner = jnp.bitwise_xor(my_pos, 1 << k)
            pl.semaphore_signal(
                barrier_sem, inc=1,
                device_id=(partner,), device_id_type=pl.DeviceIdType.MESH,
            )
        pl.semaphore_wait(barrier_sem, N_ROUNDS)

        wq = wq_ref[...]
        for b in range(B):
            xb = x_ref[b]
            qb_all = jnp.dot(xb, wq, preferred_element_type=jnp.float32)
            for h in range(Hq):
                q = qb_all[:, h * Dh:(h + 1) * Dh]
                kk = k_ref[b, :, h, :]
                vv = v_ref[b, :, h, :]
                s = lax.dot_general(
                    q, kk, (((1,), (1,)), ((), ())),
                    preferred_element_type=jnp.float32,
                ) * 0.125
                qblk = lax.broadcasted_iota(jnp.int32, (Sq, SKV_LOC), 0) // 64
                kblk = (2 * my_pos
                        + lax.broadcasted_iota(jnp.int32, (Sq, SKV_LOC), 1) // 64)
                mask = (kblk == qblk) | ((kblk % 4) == qblk)
                s = jnp.where(mask, s, NEG)
                m = jnp.max(s, axis=1)
                w = jnp.exp(s - m[:, None])
                l = jnp.sum(w, axis=1)
                a = jnp.dot(w, vv, preferred_element_type=jnp.float32)
                r = b * Hq + h
                acc_ref[r] = a
                ml_ref[0, r] = m
                ml_ref[1, r] = l

        for k in range(N_ROUNDS):
            partner = jnp.bitwise_xor(my_pos, 1 << k)
            rdma_acc = pltpu.make_async_remote_copy(
                src_ref=acc_ref, dst_ref=comm_acc.at[k],
                send_sem=send_sems.at[k, 0], recv_sem=recv_sems.at[k, 0],
                device_id=(partner,), device_id_type=pl.DeviceIdType.MESH,
            )
            rdma_ml = pltpu.make_async_remote_copy(
                src_ref=ml_ref, dst_ref=comm_ml.at[k],
                send_sem=send_sems.at[k, 1], recv_sem=recv_sems.at[k, 1],
                device_id=(partner,), device_id_type=pl.DeviceIdType.MESH,
            )
            rdma_acc.start()
            rdma_ml.start()
            rdma_acc.wait()
            rdma_ml.wait()

            m1 = ml_ref[0]
            l1 = ml_ref[1]
            m2 = comm_ml[k, 0]
            l2 = comm_ml[k, 1]
            mn = jnp.maximum(m1, m2)
            a1 = jnp.exp(m1 - mn)
            a2 = jnp.exp(m2 - mn)
            ml_ref[0] = mn
            ml_ref[1] = l1 * a1 + l2 * a2
            acc_ref[...] = (acc_ref[...] * a1[:, :, None]
                            + comm_acc[k] * a2[:, :, None])

        wo = wo_ref[...]
        for b in range(B):
            parts = []
            for h in range(Hq):
                r = b * Hq + h
                parts.append(acc_ref[r] / ml_ref[1, r][:, None])
            ctx = jnp.concatenate(parts, axis=1)
            out_ref[b] = jnp.dot(ctx, wo, preferred_element_type=jnp.float32)

        @functools.partial(pl.run_scoped, sem=pltpu.SemaphoreType.REGULAR)
        def _(sem):
            for k in range(N_ROUNDS):
                partner = jnp.bitwise_xor(my_pos, 1 << k)
                pl.semaphore_signal(
                    sem, inc=1,
                    device_id=(partner,), device_id_type=pl.DeviceIdType.MESH,
                )
            pl.semaphore_wait(sem, N_ROUNDS)

    return pl.pallas_call(
        body,
        out_shape=jax.ShapeDtypeStruct((B, Sq, Hq * Dh * 2), jnp.float32),
        in_specs=[pl.BlockSpec(memory_space=pltpu.VMEM)] * 5,
        out_specs=pl.BlockSpec(memory_space=pltpu.VMEM),
        scratch_shapes=[
            pltpu.VMEM((B * Hq, Sq, Dh), jnp.float32),
            pltpu.VMEM((2, B * Hq, Sq), jnp.float32),
            pltpu.VMEM((N_ROUNDS, B * Hq, Sq, Dh), jnp.float32),
            pltpu.VMEM((N_ROUNDS, 2, B * Hq, Sq), jnp.float32),
            pltpu.SemaphoreType.DMA((N_ROUNDS, 2)),
            pltpu.SemaphoreType.DMA((N_ROUNDS, 2)),
        ],
        compiler_params=pltpu.CompilerParams(collective_id=0),
    )(x, Wq, K_ext, V_ext, Wo)


# device time: 41809 ns/iter; 1.5202x vs baseline; 1.5202x over previous
import functools

import jax
import jax.numpy as jnp
from jax import lax
from jax.experimental import pallas as pl
from jax.experimental.pallas import tpu as pltpu

N_DEV = 32
N_ROUNDS = 5
B, Sq, Hq, Dh = 2, 128, 4, 64
SKV_LOC = 128
NEG = -1e9
R = B * Hq
PD = Dh + 2


def kernel(x, Wq, K_ext, V_ext, Wo):
    def body(x_ref, wq_ref, k_ref, v_ref, wo_ref, out_ref,
             p_ref, comm, send_sems, recv_sems):
        my_pos = lax.axis_index("i")

        barrier_sem = pltpu.get_barrier_semaphore()
        for k in range(N_ROUNDS):
            partner = jnp.bitwise_xor(my_pos, 1 << k)
            pl.semaphore_signal(
                barrier_sem, inc=1,
                device_id=(partner,), device_id_type=pl.DeviceIdType.MESH,
            )
        pl.semaphore_wait(barrier_sem, N_ROUNDS)

        wq = wq_ref[...]
        qblk = lax.broadcasted_iota(jnp.int32, (Sq, SKV_LOC), 0) // 64
        kblk = (2 * my_pos
                + lax.broadcasted_iota(jnp.int32, (Sq, SKV_LOC), 1) // 64)
        mask = (kblk == qblk) | ((kblk % 4) == qblk)
        for b in range(B):
            xb = x_ref[b]
            qb_all = jnp.dot(xb, wq, preferred_element_type=jnp.float32)
            for h in range(Hq):
                q = qb_all[:, h * Dh:(h + 1) * Dh]
                kk = k_ref[b, :, h, :]
                vv = v_ref[b, :, h, :]
                s = lax.dot_general(
                    q, kk, (((1,), (1,)), ((), ())),
                    preferred_element_type=jnp.float32,
                ) * 0.125
                s = jnp.where(mask, s, NEG)
                m = jnp.max(s, axis=1)
                w = jnp.exp(s - m[:, None])
                l = jnp.sum(w, axis=1)
                accT = lax.dot_general(
                    vv, w, (((0,), (1,)), ((), ())),
                    preferred_element_type=jnp.float32,
                )
                r = b * Hq + h
                p_ref[r, 0:Dh, :] = accT
                p_ref[r, Dh, :] = m
                p_ref[r, Dh + 1, :] = l

        for k in range(N_ROUNDS):
            partner = jnp.bitwise_xor(my_pos, 1 << k)
            rdma = pltpu.make_async_remote_copy(
                src_ref=p_ref, dst_ref=comm.at[k],
                send_sem=send_sems.at[k], recv_sem=recv_sems.at[k],
                device_id=(partner,), device_id_type=pl.DeviceIdType.MESH,
            )
            rdma.start()
            rdma.wait()

            a_1 = p_ref[:, 0:Dh, :]
            m1 = p_ref[:, Dh:Dh + 1, :]
            l1 = p_ref[:, Dh + 1:Dh + 2, :]
            a_2 = comm[k, :, 0:Dh, :]
            m2 = comm[k, :, Dh:Dh + 1, :]
            l2 = comm[k, :, Dh + 1:Dh + 2, :]
            mn = jnp.maximum(m1, m2)
            e1 = jnp.exp(m1 - mn)
            e2 = jnp.exp(m2 - mn)
            p_ref[:, 0:Dh, :] = a_1 * e1 + a_2 * e2
            p_ref[:, Dh:Dh + 1, :] = mn
            p_ref[:, Dh + 1:Dh + 2, :] = l1 * e1 + l2 * e2

        for b in range(B):
            out_b = None
            for h in range(Hq):
                r = b * Hq + h
                ctxT = p_ref[r, 0:Dh, :] / p_ref[r, Dh + 1:Dh + 2, :]
                wo_h = wo_ref[h * Dh:(h + 1) * Dh, :]
                term = lax.dot_general(
                    ctxT, wo_h, (((0,), (0,)), ((), ())),
                    preferred_element_type=jnp.float32,
                )
                out_b = term if out_b is None else out_b + term
            out_ref[b] = out_b

        @functools.partial(pl.run_scoped, sem=pltpu.SemaphoreType.REGULAR)
        def _(sem):
            for k in range(N_ROUNDS):
                partner = jnp.bitwise_xor(my_pos, 1 << k)
                pl.semaphore_signal(
                    sem, inc=1,
                    device_id=(partner,), device_id_type=pl.DeviceIdType.MESH,
                )
            pl.semaphore_wait(sem, N_ROUNDS)

    return pl.pallas_call(
        body,
        out_shape=jax.ShapeDtypeStruct((B, Sq, 512), jnp.float32),
        in_specs=[pl.BlockSpec(memory_space=pltpu.VMEM)] * 5,
        out_specs=pl.BlockSpec(memory_space=pltpu.VMEM),
        scratch_shapes=[
            pltpu.VMEM((R, PD, Sq), jnp.float32),
            pltpu.VMEM((N_ROUNDS, R, PD, Sq), jnp.float32),
            pltpu.SemaphoreType.DMA((N_ROUNDS,)),
            pltpu.SemaphoreType.DMA((N_ROUNDS,)),
        ],
        compiler_params=pltpu.CompilerParams(collective_id=0),
    )(x, Wq, K_ext, V_ext, Wo)
